# baseline (speedup 1.0000x reference)
import functools

import jax
import jax.numpy as jnp
from jax import lax
from jax.experimental import pallas as pl
from jax.experimental.pallas import tpu as pltpu
from jax.experimental.pallas import tpu_sc as plsc

N_NODES = 10000
N_EDGES = 320000
IN_FEATS = 128
MLP_HIDDEN = 64
PROJ_FEATS = 64

NCORES = 2
NSUB = 16
HALF = N_NODES // NCORES
ACCR = 5120
G = 16
E_PAD = 327680
EPT = E_PAD // NSUB
GPT = EPT // G
GPB = 256
NBB = GPT // GPB
DEPTH = 2
NPT = ACCR // NSUB

_mesh = plsc.VectorSubcoreMesh(core_axis_name="c", subcore_axis_name="s")


def _sc_degrees_body(src_hbm, dst_hbm, out_hbm, sidx, didx, ones_s, ones_d, sem, acc):
    cid = lax.axis_index("c")
    sid = lax.axis_index("s")
    base = cid * HALF

    def fillz(i, carry):
        for k in range(8):
            ones_s[i, pl.ds(k * 16, 16)] = jnp.zeros((16,), jnp.float32)
        return carry

    lax.fori_loop(0, G, fillz, 0)

    def zero_chunk(k, carry):
        c = sid + k * NSUB
        pltpu.sync_copy(ones_s, acc.at[pl.ds(c * G, G)])
        return carry

    lax.fori_loop(0, ACCR // G // NSUB, zero_chunk, 0)

    def fill1(i, carry):
        for k in range(4):
            ones_s[i, pl.ds(k * 16, 16)] = jnp.full((16,), 1.0, jnp.float32)
            ones_d[i, pl.ds(k * 16, 16)] = jnp.zeros((16,), jnp.float32)
        for k in range(4, 8):
            ones_s[i, pl.ds(k * 16, 16)] = jnp.zeros((16,), jnp.float32)
            ones_d[i, pl.ds(k * 16, 16)] = jnp.full((16,), 1.0, jnp.float32)
        return carry

    lax.fori_loop(0, G, fill1, 0)
    plsc.subcore_barrier()

    def block(b, carry):
        pltpu.sync_copy(src_hbm.at[sid, b], sidx)
        pltpu.sync_copy(dst_hbm.at[sid, b], didx)

        def grp(p, carry2):
            iv_s = sidx[p] - base
            iv_s = jnp.where((iv_s >= 0) & (iv_s < HALF), iv_s, HALF)
            iv_d = didx[p] - base
            iv_d = jnp.where((iv_d >= 0) & (iv_d < HALF), iv_d, HALF)
            pltpu.async_copy(ones_s, acc.at[iv_s], sem, add=True)
            pltpu.async_copy(ones_d, acc.at[iv_d], sem, add=True)

            @pl.when(p >= 2)
            def _():
                pltpu.make_async_copy(ones_s, acc.at[iv_s], sem).wait()
                pltpu.make_async_copy(ones_d, acc.at[iv_d], sem).wait()

            return carry2

        lax.fori_loop(0, GPB, grp, 0)
        for _ in range(2):
            pltpu.make_async_copy(ones_s, acc.at[pl.ds(0, G)], sem).wait()
            pltpu.make_async_copy(ones_d, acc.at[pl.ds(0, G)], sem).wait()
        return carry

    lax.fori_loop(0, NBB, block, 0)
    plsc.subcore_barrier()
    pltpu.sync_copy(acc.at[pl.ds(sid * NPT, NPT)], out_hbm.at[cid, sid])


def _sc_segsum_body(src_hbm, dst_hbm, g_hbm, out_hbm, sidx, didx, bufs_v,
                    sem0, sem1, sem2, sem3, sems, acc):
    cid = lax.axis_index("c")
    sid = lax.axis_index("s")
    base = cid * HALF
    gsems = [sem0, sem1, sem2, sem3][:DEPTH]

    def fillz(i, carry):
        for k in range(8):
            bufs_v[0, i, pl.ds(k * 16, 16)] = jnp.zeros((16,), jnp.float32)
        return carry

    lax.fori_loop(0, G, fillz, 0)

    def zero_chunk(k, carry):
        c = sid + k * NSUB
        pltpu.sync_copy(bufs_v.at[0], acc.at[pl.ds(c * G, G)])
        return carry

    lax.fori_loop(0, ACCR // G // NSUB, zero_chunk, 0)
    plsc.subcore_barrier()

    def gclamp(p):
        return jnp.minimum(sidx[p], N_NODES - 1)

    def block(b, carry):
        pltpu.sync_copy(src_hbm.at[sid, b], sidx)
        pltpu.sync_copy(dst_hbm.at[sid, b], didx)

        for i in range(DEPTH):
            pltpu.async_copy(g_hbm.at[gclamp(i)], bufs_v.at[i], gsems[i])

        def quad(t, carry2):
            p0 = DEPTH * t
            for i in range(DEPTH):
                p = p0 + i
                buf = bufs_v.at[i]
                pltpu.make_async_copy(g_hbm.at[gclamp(p)], buf, gsems[i]).wait()
                iv_d = didx[p] - base
                iv_d = jnp.where((iv_d >= 0) & (iv_d < HALF), iv_d, HALF)
                pltpu.async_copy(buf, acc.at[iv_d], sems, add=True)

                @pl.when(p + DEPTH < GPB)
                def _():
                    pltpu.make_async_copy(buf, acc.at[iv_d], sems).wait()
                    pltpu.async_copy(g_hbm.at[gclamp(p + DEPTH)], buf, gsems[i])

                @pl.when(p + DEPTH >= GPB)
                def _():
                    pltpu.make_async_copy(buf, acc.at[iv_d], sems).wait()

            return carry2

        lax.fori_loop(0, GPB // DEPTH, quad, 0)
        return carry

    lax.fori_loop(0, NBB, block, 0)
    plsc.subcore_barrier()
    pltpu.sync_copy(acc.at[pl.ds(sid * NPT, NPT)], out_hbm.at[cid, sid])


def _make_sc_kernels(interpret=False):
    deg = functools.partial(
        pl.kernel,
        out_type=jax.ShapeDtypeStruct((NCORES, NSUB, NPT, 128), jnp.float32),
        mesh=_mesh,
        scratch_types=[
            pltpu.VMEM((GPB, G), jnp.int32),
            pltpu.VMEM((GPB, G), jnp.int32),
            pltpu.VMEM((G, 128), jnp.float32),
            pltpu.VMEM((G, 128), jnp.float32),
            pltpu.SemaphoreType.DMA,
            pltpu.VMEM_SHARED((ACCR, 128), jnp.float32),
        ],
        interpret=interpret,
    )(_sc_degrees_body)
    seg = functools.partial(
        pl.kernel,
        out_type=jax.ShapeDtypeStruct((NCORES, NSUB, NPT, 128), jnp.float32),
        mesh=_mesh,
        scratch_types=[
            pltpu.VMEM((GPB, G), jnp.int32),
            pltpu.VMEM((GPB, G), jnp.int32),
            pltpu.VMEM((DEPTH, G, 128), jnp.float32),
            pltpu.SemaphoreType.DMA,
            pltpu.SemaphoreType.DMA,
            pltpu.SemaphoreType.DMA,
            pltpu.SemaphoreType.DMA,
            pltpu.SemaphoreType.DMA,
            pltpu.VMEM_SHARED((ACCR, 128), jnp.float32),
        ],
        interpret=interpret,
    )(_sc_segsum_body)
    return deg, seg


_sc_degrees, _sc_segsum = _make_sc_kernels()


def _tc_g_body(x_ref, degp_ref, W1_ref, g_ref):
    deg = jnp.concatenate(
        [degp_ref[0, :HALF, 0:1], degp_ref[1, :HALF, 0:1]], axis=0)
    ns = jnp.where(deg > 0, lax.rsqrt(deg), 0.0)
    g_ref[...] = jnp.dot(x_ref[...] * ns, W1_ref[...],
                         precision=lax.Precision.HIGHEST)


def _tc_head_body(segp_ref, degp_ref, b1_ref, Wp1_ref, bp1_ref, gamma_ref,
                  beta_ref, Wp2_ref, bp2_ref, out_ref):
    seg = jnp.concatenate([segp_ref[0, :HALF, :], segp_ref[1, :HALF, :]], axis=0)
    deg = jnp.concatenate(
        [degp_ref[0, :HALF, 64:65], degp_ref[1, :HALF, 64:65]], axis=0)
    nd = jnp.where(deg > 0, lax.rsqrt(deg), 0.0)
    c = jnp.dot(b1_ref[...], Wp1_ref[...],
                precision=lax.Precision.HIGHEST) + bp1_ref[...]
    z = jnp.dot(seg * nd, Wp1_ref[...], precision=lax.Precision.HIGHEST) + c
    mean = jnp.mean(z, axis=0, keepdims=True)
    var = jnp.mean((z - mean) ** 2, axis=0, keepdims=True)
    z = (z - mean) / jnp.sqrt(var + 1e-5) * gamma_ref[...] + beta_ref[...]
    z = jnp.maximum(z, 0.0)
    out_ref[...] = jnp.dot(z, Wp2_ref[...],
                           precision=lax.Precision.HIGHEST) + bp2_ref[...]


def kernel(x, edge_index, W1, b1, Wp1, bp1, gamma, beta, Wp2, bp2):
    ep = jnp.pad(edge_index, ((0, 0), (0, E_PAD - N_EDGES)),
                 constant_values=N_NODES)
    src4 = ep[0].reshape(NSUB, NBB, GPB, G)
    dst4 = ep[1].reshape(NSUB, NBB, GPB, G)

    degp = _sc_degrees(src4, dst4).reshape(NCORES, ACCR, 128)

    g = pl.pallas_call(
        _tc_g_body,
        out_shape=jax.ShapeDtypeStruct((N_NODES, IN_FEATS), jnp.float32),
    )(x, degp, W1)

    segp = _sc_segsum(src4, dst4, g).reshape(NCORES, ACCR, 128)

    out = pl.pallas_call(
        _tc_head_body,
        out_shape=jax.ShapeDtypeStruct((N_NODES, PROJ_FEATS), jnp.float32),
    )(segp, degp, b1.reshape(1, IN_FEATS), Wp1, bp1.reshape(1, MLP_HIDDEN),
      gamma.reshape(1, MLP_HIDDEN), beta.reshape(1, MLP_HIDDEN), Wp2,
      bp2.reshape(1, PROJ_FEATS))
    return out


# speedup vs baseline: 1.4369x; 1.4369x over previous
import functools

import jax
import jax.numpy as jnp
from jax import lax
from jax.experimental import pallas as pl
from jax.experimental.pallas import tpu as pltpu
from jax.experimental.pallas import tpu_sc as plsc

N_NODES = 10000
N_EDGES = 320000
IN_FEATS = 128
MLP_HIDDEN = 64
PROJ_FEATS = 64

NCORES = 2
NSUB = 16
HALF = N_NODES // NCORES
ACCR = 5120
G = 16
EPT = N_EDGES // NSUB
GPT = EPT // G
GPB = 250
NBB = GPT // GPB
NPT = ACCR // NSUB

_mesh = plsc.VectorSubcoreMesh(core_axis_name="c", subcore_axis_name="s")


def _sc_degrees_body(src_hbm, dst_hbm, out_hbm, sidx, didx, ones_s, ones_d, sem, acc):
    cid = lax.axis_index("c")
    sid = lax.axis_index("s")
    base = cid * HALF

    def fillz(i, carry):
        for k in range(8):
            ones_s[i, pl.ds(k * 16, 16)] = jnp.zeros((16,), jnp.float32)
        return carry

    lax.fori_loop(0, G, fillz, 0)

    def zero_chunk(k, carry):
        c = sid + k * NSUB
        pltpu.sync_copy(ones_s, acc.at[pl.ds(c * G, G)])
        return carry

    lax.fori_loop(0, ACCR // G // NSUB, zero_chunk, 0)

    def fill1(i, carry):
        for k in range(4):
            ones_s[i, pl.ds(k * 16, 16)] = jnp.full((16,), 1.0, jnp.float32)
            ones_d[i, pl.ds(k * 16, 16)] = jnp.zeros((16,), jnp.float32)
        for k in range(4, 8):
            ones_s[i, pl.ds(k * 16, 16)] = jnp.zeros((16,), jnp.float32)
            ones_d[i, pl.ds(k * 16, 16)] = jnp.full((16,), 1.0, jnp.float32)
        return carry

    lax.fori_loop(0, G, fill1, 0)
    plsc.subcore_barrier()

    def block(b, carry):
        pltpu.sync_copy(src_hbm.at[sid, b], sidx)
        pltpu.sync_copy(dst_hbm.at[sid, b], didx)

        def grp(p, carry2):
            iv_s = sidx[p] - base
            iv_s = jnp.where((iv_s >= 0) & (iv_s < HALF), iv_s, HALF)
            iv_d = didx[p] - base
            iv_d = jnp.where((iv_d >= 0) & (iv_d < HALF), iv_d, HALF)
            pltpu.async_copy(ones_s, acc.at[iv_s], sem, add=True)
            pltpu.async_copy(ones_d, acc.at[iv_d], sem, add=True)

            @pl.when(p >= 2)
            def _():
                pltpu.make_async_copy(ones_s, acc.at[iv_s], sem).wait()
                pltpu.make_async_copy(ones_d, acc.at[iv_d], sem).wait()

            return carry2

        lax.fori_loop(0, GPB, grp, 0)
        for _ in range(2):
            pltpu.make_async_copy(ones_s, acc.at[pl.ds(0, G)], sem).wait()
            pltpu.make_async_copy(ones_d, acc.at[pl.ds(0, G)], sem).wait()
        return carry

    lax.fori_loop(0, NBB, block, 0)
    plsc.subcore_barrier()
    pltpu.sync_copy(acc.at[pl.ds(sid * NPT, NPT)], out_hbm.at[cid, sid])


def _sc_segsum_body(src_hbm, dst_hbm, g_hbm, out_hbm, sidx, didx, bufa, bufb,
                    sema, semb, sems, acc):
    cid = lax.axis_index("c")
    sid = lax.axis_index("s")
    base = cid * HALF

    def fillz(i, carry):
        for k in range(8):
            bufa[i, pl.ds(k * 16, 16)] = jnp.zeros((16,), jnp.float32)
        return carry

    lax.fori_loop(0, G, fillz, 0)

    def zero_chunk(k, carry):
        c = sid + k * NSUB
        pltpu.sync_copy(bufa, acc.at[pl.ds(c * G, G)])
        return carry

    lax.fori_loop(0, ACCR // G // NSUB, zero_chunk, 0)
    plsc.subcore_barrier()

    def block(b, carry):
        pltpu.sync_copy(src_hbm.at[sid, b], sidx)
        pltpu.sync_copy(dst_hbm.at[sid, b], didx)

        pltpu.async_copy(g_hbm.at[sidx[0]], bufa, sema)
        pltpu.async_copy(g_hbm.at[sidx[1]], bufb, semb)

        def pair(t, carry2):
            p0 = 2 * t
            for off, buf, sem in ((0, bufa, sema), (1, bufb, semb)):
                p = p0 + off
                pltpu.make_async_copy(g_hbm.at[sidx[p]], buf, sem).wait()
                iv_d = didx[p] - base
                iv_d = jnp.where((iv_d >= 0) & (iv_d < HALF), iv_d, HALF)
                pltpu.async_copy(buf, acc.at[iv_d], sems, add=True)

                @pl.when(p + 2 < GPB)
                def _():
                    pltpu.make_async_copy(buf, acc.at[iv_d], sems).wait()
                    pltpu.async_copy(g_hbm.at[sidx[p + 2]], buf, sem)

                @pl.when(p + 2 >= GPB)
                def _():
                    pltpu.make_async_copy(buf, acc.at[iv_d], sems).wait()

            return carry2

        lax.fori_loop(0, GPB // 2, pair, 0)
        return carry

    lax.fori_loop(0, NBB, block, 0)
    plsc.subcore_barrier()
    pltpu.sync_copy(acc.at[pl.ds(sid * NPT, NPT)], out_hbm.at[cid, sid])


def _make_sc_kernels(interpret=False):
    deg = functools.partial(
        pl.kernel,
        out_type=jax.ShapeDtypeStruct((NCORES, NSUB, NPT, 128), jnp.float32),
        mesh=_mesh,
        scratch_types=[
            pltpu.VMEM((GPB, G), jnp.int32),
            pltpu.VMEM((GPB, G), jnp.int32),
            pltpu.VMEM((G, 128), jnp.float32),
            pltpu.VMEM((G, 128), jnp.float32),
            pltpu.SemaphoreType.DMA,
            pltpu.VMEM_SHARED((ACCR, 128), jnp.float32),
        ],
        interpret=interpret,
    )(_sc_degrees_body)
    seg = functools.partial(
        pl.kernel,
        out_type=jax.ShapeDtypeStruct((NCORES, NSUB, NPT, 128), jnp.float32),
        mesh=_mesh,
        scratch_types=[
            pltpu.VMEM((GPB, G), jnp.int32),
            pltpu.VMEM((GPB, G), jnp.int32),
            pltpu.VMEM((G, 128), jnp.float32),
            pltpu.VMEM((G, 128), jnp.float32),
            pltpu.SemaphoreType.DMA,
            pltpu.SemaphoreType.DMA,
            pltpu.SemaphoreType.DMA,
            pltpu.VMEM_SHARED((ACCR, 128), jnp.float32),
        ],
        interpret=interpret,
    )(_sc_segsum_body)
    return deg, seg


_sc_degrees, _sc_segsum = _make_sc_kernels()


def _tc_g_body(x_ref, degp_ref, W1_ref, g_ref):
    deg = jnp.concatenate(
        [degp_ref[0, :HALF, 0:1], degp_ref[1, :HALF, 0:1]], axis=0)
    ns = jnp.where(deg > 0, lax.rsqrt(deg), 0.0)
    g_ref[...] = jnp.dot(x_ref[...] * ns, W1_ref[...],
                         precision=lax.Precision.HIGHEST)


def _tc_head_body(segp_ref, degp_ref, b1_ref, Wp1_ref, bp1_ref, gamma_ref,
                  beta_ref, Wp2_ref, bp2_ref, out_ref):
    seg = jnp.concatenate([segp_ref[0, :HALF, :], segp_ref[1, :HALF, :]], axis=0)
    deg = jnp.concatenate(
        [degp_ref[0, :HALF, 64:65], degp_ref[1, :HALF, 64:65]], axis=0)
    nd = jnp.where(deg > 0, lax.rsqrt(deg), 0.0)
    c = jnp.dot(b1_ref[...], Wp1_ref[...],
                precision=lax.Precision.HIGHEST) + bp1_ref[...]
    z = jnp.dot(seg * nd, Wp1_ref[...], precision=lax.Precision.HIGHEST) + c
    mean = jnp.mean(z, axis=0, keepdims=True)
    var = jnp.mean((z - mean) ** 2, axis=0, keepdims=True)
    z = (z - mean) / jnp.sqrt(var + 1e-5) * gamma_ref[...] + beta_ref[...]
    z = jnp.maximum(z, 0.0)
    out_ref[...] = jnp.dot(z, Wp2_ref[...],
                           precision=lax.Precision.HIGHEST) + bp2_ref[...]


def kernel(x, edge_index, W1, b1, Wp1, bp1, gamma, beta, Wp2, bp2):
    src4 = edge_index[0].reshape(NSUB, NBB, GPB, G)
    dst4 = edge_index[1].reshape(NSUB, NBB, GPB, G)

    degp = _sc_degrees(src4, dst4).reshape(NCORES, ACCR, 128)

    g = pl.pallas_call(
        _tc_g_body,
        out_shape=jax.ShapeDtypeStruct((N_NODES, IN_FEATS), jnp.float32),
    )(x, degp, W1)

    segp = _sc_segsum(src4, dst4, g).reshape(NCORES, ACCR, 128)

    out = pl.pallas_call(
        _tc_head_body,
        out_shape=jax.ShapeDtypeStruct((N_NODES, PROJ_FEATS), jnp.float32),
    )(segp, degp, b1.reshape(1, IN_FEATS), Wp1, bp1.reshape(1, MLP_HIDDEN),
      gamma.reshape(1, MLP_HIDDEN), beta.reshape(1, MLP_HIDDEN), Wp2,
      bp2.reshape(1, PROJ_FEATS))
    return out


# speedup vs baseline: 1.4372x; 1.0002x over previous
import functools

import jax
import jax.numpy as jnp
from jax import lax
from jax.experimental import pallas as pl
from jax.experimental.pallas import tpu as pltpu
from jax.experimental.pallas import tpu_sc as plsc

N_NODES = 10000
N_EDGES = 320000
IN_FEATS = 128
MLP_HIDDEN = 64
PROJ_FEATS = 64

NCORES = 2
NSUB = 16
HALF = N_NODES // NCORES
ACCR = 5120
G = 16
EPT = N_EDGES // NSUB
GPT = EPT // G
GPB = 250
NBB = GPT // GPB
NPT = ACCR // NSUB

_mesh = plsc.VectorSubcoreMesh(core_axis_name="c", subcore_axis_name="s")


def _sc_degrees_body(src_hbm, dst_hbm, out_hbm, sidx, didx, ones_s, ones_d, sem, acc):
    cid = lax.axis_index("c")
    sid = lax.axis_index("s")
    base = cid * HALF

    def fillz(i, carry):
        for k in range(8):
            ones_s[i, pl.ds(k * 16, 16)] = jnp.zeros((16,), jnp.float32)
        return carry

    lax.fori_loop(0, G, fillz, 0)

    def zero_chunk(k, carry):
        c = sid + k * NSUB
        pltpu.sync_copy(ones_s, acc.at[pl.ds(c * G, G)])
        return carry

    lax.fori_loop(0, ACCR // G // NSUB, zero_chunk, 0)

    def fill1(i, carry):
        for k in range(4):
            ones_s[i, pl.ds(k * 16, 16)] = jnp.full((16,), 1.0, jnp.float32)
            ones_d[i, pl.ds(k * 16, 16)] = jnp.zeros((16,), jnp.float32)
        for k in range(4, 8):
            ones_s[i, pl.ds(k * 16, 16)] = jnp.zeros((16,), jnp.float32)
            ones_d[i, pl.ds(k * 16, 16)] = jnp.full((16,), 1.0, jnp.float32)
        return carry

    lax.fori_loop(0, G, fill1, 0)
    plsc.subcore_barrier()

    def block(b, carry):
        pltpu.sync_copy(src_hbm.at[sid, b], sidx)
        pltpu.sync_copy(dst_hbm.at[sid, b], didx)

        def grp(p, carry2):
            iv_s = sidx[p] - base
            iv_s = jnp.where((iv_s >= 0) & (iv_s < HALF), iv_s, HALF)
            iv_d = didx[p] - base
            iv_d = jnp.where((iv_d >= 0) & (iv_d < HALF), iv_d, HALF)
            pltpu.async_copy(ones_s, acc.at[iv_s], sem, add=True)
            pltpu.async_copy(ones_d, acc.at[iv_d], sem, add=True)

            @pl.when(p >= 2)
            def _():
                pltpu.make_async_copy(ones_s, acc.at[iv_s], sem).wait()
                pltpu.make_async_copy(ones_d, acc.at[iv_d], sem).wait()

            return carry2

        lax.fori_loop(0, GPB, grp, 0)
        for _ in range(2):
            pltpu.make_async_copy(ones_s, acc.at[pl.ds(0, G)], sem).wait()
            pltpu.make_async_copy(ones_d, acc.at[pl.ds(0, G)], sem).wait()
        return carry

    lax.fori_loop(0, NBB, block, 0)
    plsc.subcore_barrier()
    pltpu.sync_copy(acc.at[pl.ds(sid * NPT, NPT)], out_hbm.at[cid, sid])


def _sc_segsum_body(src_hbm, dst_hbm, g_hbm, out_hbm, sidx, didx, bufa, bufb,
                    sema, semb, sems, acc):
    cid = lax.axis_index("c")
    sid = lax.axis_index("s")
    base = cid * HALF

    def fillz(i, carry):
        for k in range(8):
            bufa[i, pl.ds(k * 16, 16)] = jnp.zeros((16,), jnp.float32)
        return carry

    lax.fori_loop(0, G, fillz, 0)

    def zero_chunk(k, carry):
        c = sid + k * NSUB
        pltpu.sync_copy(bufa, acc.at[pl.ds(c * G, G)])
        return carry

    lax.fori_loop(0, ACCR // G // NSUB, zero_chunk, 0)
    plsc.subcore_barrier()

    def block(b, carry):
        pltpu.sync_copy(src_hbm.at[sid, b], sidx)
        pltpu.sync_copy(dst_hbm.at[sid, b], didx)

        pltpu.async_copy(g_hbm.at[sidx[0]], bufa, sema)
        pltpu.async_copy(g_hbm.at[sidx[1]], bufb, semb)

        def pair(t, carry2):
            p0 = 2 * t
            for off, buf, sem in ((0, bufa, sema), (1, bufb, semb)):
                p = p0 + off
                pltpu.make_async_copy(g_hbm.at[sidx[p]], buf, sem).wait()
                iv_d = didx[p] - base
                iv_d = jnp.where((iv_d >= 0) & (iv_d < HALF), iv_d, HALF)
                pltpu.async_copy(buf, acc.at[iv_d], sems, add=True)

                @pl.when(p + 2 < GPB)
                def _():
                    pltpu.make_async_copy(buf, acc.at[iv_d], sems).wait()
                    pltpu.async_copy(g_hbm.at[sidx[p + 2]], buf, sem)

                @pl.when(p + 2 >= GPB)
                def _():
                    pltpu.make_async_copy(buf, acc.at[iv_d], sems).wait()

            return carry2

        lax.fori_loop(0, GPB // 2, pair, 0)
        return carry

    lax.fori_loop(0, NBB, block, 0)
    plsc.subcore_barrier()
    pltpu.sync_copy(acc.at[pl.ds(sid * NPT, NPT)], out_hbm.at[cid, sid])


def _make_sc_kernels():
    deg = functools.partial(
        pl.kernel,
        out_type=jax.ShapeDtypeStruct((NCORES, NSUB, NPT, 128), jnp.float32),
        mesh=_mesh,
        scratch_types=[
            pltpu.VMEM((GPB, G), jnp.int32),
            pltpu.VMEM((GPB, G), jnp.int32),
            pltpu.VMEM((G, 128), jnp.float32),
            pltpu.VMEM((G, 128), jnp.float32),
            pltpu.SemaphoreType.DMA,
            pltpu.VMEM_SHARED((ACCR, 128), jnp.float32),
        ],
    )(_sc_degrees_body)
    seg = functools.partial(
        pl.kernel,
        out_type=jax.ShapeDtypeStruct((NCORES, NSUB, NPT, 128), jnp.float32),
        mesh=_mesh,
        scratch_types=[
            pltpu.VMEM((GPB, G), jnp.int32),
            pltpu.VMEM((GPB, G), jnp.int32),
            pltpu.VMEM((G, 128), jnp.float32),
            pltpu.VMEM((G, 128), jnp.float32),
            pltpu.SemaphoreType.DMA,
            pltpu.SemaphoreType.DMA,
            pltpu.SemaphoreType.DMA,
            pltpu.VMEM_SHARED((ACCR, 128), jnp.float32),
        ],
    )(_sc_segsum_body)
    return deg, seg


_sc_degrees, _sc_segsum = _make_sc_kernels()


def _tc_g_body(x_ref, degp_ref, W1_ref, g_ref):
    deg = jnp.concatenate(
        [degp_ref[0, :HALF, 0:1], degp_ref[1, :HALF, 0:1]], axis=0)
    ns = jnp.where(deg > 0, lax.rsqrt(deg), 0.0)
    g_ref[...] = jnp.dot(x_ref[...] * ns, W1_ref[...],
                         precision=lax.Precision.HIGHEST)


def _tc_head_body(segp_ref, degp_ref, b1_ref, Wp1_ref, bp1_ref, gamma_ref,
                  beta_ref, Wp2_ref, bp2_ref, out_ref):
    seg = jnp.concatenate([segp_ref[0, :HALF, :], segp_ref[1, :HALF, :]], axis=0)
    deg = jnp.concatenate(
        [degp_ref[0, :HALF, 64:65], degp_ref[1, :HALF, 64:65]], axis=0)
    nd = jnp.where(deg > 0, lax.rsqrt(deg), 0.0)
    c = jnp.dot(b1_ref[...], Wp1_ref[...],
                precision=lax.Precision.HIGHEST) + bp1_ref[...]
    z = jnp.dot(seg * nd, Wp1_ref[...], precision=lax.Precision.HIGHEST) + c
    mean = jnp.mean(z, axis=0, keepdims=True)
    var = jnp.mean((z - mean) ** 2, axis=0, keepdims=True)
    z = (z - mean) / jnp.sqrt(var + 1e-5) * gamma_ref[...] + beta_ref[...]
    z = jnp.maximum(z, 0.0)
    out_ref[...] = jnp.dot(z, Wp2_ref[...],
                           precision=lax.Precision.HIGHEST) + bp2_ref[...]


def kernel(x, edge_index, W1, b1, Wp1, bp1, gamma, beta, Wp2, bp2):
    src4 = edge_index[0].reshape(NSUB, NBB, GPB, G)
    dst4 = edge_index[1].reshape(NSUB, NBB, GPB, G)

    degp = _sc_degrees(src4, dst4).reshape(NCORES, ACCR, 128)

    g = pl.pallas_call(
        _tc_g_body,
        out_shape=jax.ShapeDtypeStruct((N_NODES, IN_FEATS), jnp.float32),
    )(x, degp, W1)

    segp = _sc_segsum(src4, dst4, g).reshape(NCORES, ACCR, 128)

    out = pl.pallas_call(
        _tc_head_body,
        out_shape=jax.ShapeDtypeStruct((N_NODES, PROJ_FEATS), jnp.float32),
    )(segp, degp, b1.reshape(1, IN_FEATS), Wp1, bp1.reshape(1, MLP_HIDDEN),
      gamma.reshape(1, MLP_HIDDEN), beta.reshape(1, MLP_HIDDEN), Wp2,
      bp2.reshape(1, PROJ_FEATS))
    return out
